# baseline (speedup 1.0000x reference)
import functools

import jax
import jax.numpy as jnp
from jax import lax
from jax.experimental import pallas as pl
from jax.experimental.pallas import tpu as pltpu
from jax.experimental.pallas import tpu_sc as plsc

D_EMB = 1024
D_MODEL = 2048
B_DIM = 4
L_DIM = 4096
N_TOK = B_DIM * L_DIM

NC = 2
NS = 16
NW = NC * NS
TW = N_TOK // NW
K = 4
BPW = TW // K
NBLK = N_TOK // K

DBJ_EXT = (0, 0, 0, 0, 1, 0, 0, 1, 1, 0, 1, 0, 1, 1, 1, 1, 0, 0, 0)
NROWS = len(DBJ_EXT)
DBJ_LUT = (0, 1, 2, 5, 3, 9, 6, 11, 15, 4, 8, 10, 14, 7, 13, 12)


def _tc_body(x_ref, emb_ref, w_ref, b_ref, seq_ref, lut_ref, dbj_ref, woff_ref):
    proj = (
        jnp.dot(emb_ref[...], w_ref[...], preferred_element_type=jnp.float32)
        + b_ref[...][None, :]
    )
    p0 = proj[0]
    d = proj[1] - proj[0]
    seqv = seq_ref[...].reshape(NROWS, 1)
    dbj_ref[...] = p0[None, :] + seqv * d[None, :]

    xb = x_ref[...].reshape(B_DIM, L_DIM // K, K).astype(jnp.float32)
    pid = xb[..., 0] * 8.0 + xb[..., 1] * 4.0 + xb[..., 2] * 2.0 + xb[..., 3]
    lutf = lut_ref[...].reshape(1, 1, 16)
    iot = lax.broadcasted_iota(jnp.int32, (1, 1, 16), 2).astype(jnp.float32)
    onehot = pid[..., None] == iot
    woff = jnp.sum(jnp.where(onehot, lutf, 0.0), axis=-1)
    woff_ref[...] = woff.astype(jnp.int32)


def _tc_prep(x, emb_table, W, b):
    seq = jnp.array(DBJ_EXT, dtype=jnp.float32)
    lut = jnp.array(DBJ_LUT, dtype=jnp.float32)
    return pl.pallas_call(
        _tc_body,
        out_shape=[
            jax.ShapeDtypeStruct((NROWS, D_MODEL), jnp.float32),
            jax.ShapeDtypeStruct((B_DIM, L_DIM // K), jnp.int32),
        ],
    )(x, emb_table, W, b, seq, lut)


@functools.partial(
    pl.kernel,
    out_type=jax.ShapeDtypeStruct((N_TOK * D_MODEL,), jnp.float32),
    mesh=plsc.VectorSubcoreMesh(core_axis_name="c", subcore_axis_name="s"),
    compiler_params=pltpu.CompilerParams(needs_layout_passes=False),
    scratch_types=[
        pltpu.VMEM((BPW,), jnp.int32),
        pltpu.VMEM((NROWS * D_MODEL,), jnp.float32),
        pltpu.VMEM((K * D_MODEL,), jnp.float32),
        pltpu.SemaphoreType.DMA,
    ],
)
def _sc_emit(woff_hbm, dbj_hbm, out_hbm, wv, dbj_v, dbuf, sem):
    cid = lax.axis_index("c")
    sid = lax.axis_index("s")
    wid = sid * NC + cid
    blk0 = wid * BPW
    pltpu.sync_copy(woff_hbm.at[pl.ds(blk0, BPW)], wv)
    pltpu.sync_copy(dbj_hbm, dbj_v)

    GRP = 16
    NGRP = BPW // GRP
    LAG = 4

    def drain_grp():
        def one(i, carry):
            pltpu.make_async_copy(
                out_hbm.at[pl.ds(blk0 * K * D_MODEL, K * D_MODEL)], dbuf, sem
            ).wait()
            return carry

        lax.fori_loop(0, GRP, one, 0)

    def grp(g, carry):
        v = wv[pl.ds(g * GRP, GRP)]
        for i in range(GRP):
            blk = blk0 + g * GRP + i
            src = v[i] * D_MODEL
            pltpu.async_copy(
                dbj_v.at[pl.ds(src, K * D_MODEL)],
                out_hbm.at[pl.ds(blk * K * D_MODEL, K * D_MODEL)],
                sem,
            )

        @pl.when(g >= LAG)
        def _():
            drain_grp()

        return carry

    lax.fori_loop(0, NGRP, grp, 0)

    def tail(g, carry):
        drain_grp()
        return carry

    lax.fori_loop(0, LAG, tail, 0)


def kernel(x, emb_table, W, b):
    xi = x.astype(jnp.int32)
    dbj, woff = _tc_prep(xi, emb_table, W, b)
    out = _sc_emit(woff.reshape(-1), dbj.reshape(-1))
    return out.reshape(B_DIM, L_DIM, D_MODEL)


# speedup vs baseline: 2.2589x; 2.2589x over previous
import functools

import jax
import jax.numpy as jnp
from jax import lax
from jax.experimental import pallas as pl
from jax.experimental.pallas import tpu as pltpu
from jax.experimental.pallas import tpu_sc as plsc

D_EMB = 1024
D_MODEL = 2048
B_DIM = 4
L_DIM = 4096
N_TOK = B_DIM * L_DIM

NC = 2
NS = 16
NW = NC * NS
TW = N_TOK // NW
C = 16
NCHUNK = TW // C
NBUF = 2
NG = NCHUNK // NBUF


def _proj_body(emb_ref, w_ref, b_ref, out_ref):
    out_ref[...] = (
        jnp.dot(emb_ref[...], w_ref[...], preferred_element_type=jnp.float32)
        + b_ref[...][None, :]
    )


def _compute_proj(emb_table, W, b):
    return pl.pallas_call(
        _proj_body,
        out_shape=jax.ShapeDtypeStruct((2, D_MODEL), jnp.float32),
    )(emb_table, W, b)


@functools.partial(
    pl.kernel,
    out_type=jax.ShapeDtypeStruct((N_TOK, D_MODEL), jnp.float32),
    mesh=plsc.VectorSubcoreMesh(core_axis_name="c", subcore_axis_name="s"),
    compiler_params=pltpu.CompilerParams(needs_layout_passes=False),
    scratch_types=[
        pltpu.VMEM((TW,), jnp.int32),
        pltpu.VMEM((2 * D_MODEL,), jnp.float32),
        pltpu.VMEM((C, D_MODEL), jnp.float32),
        pltpu.VMEM((C, D_MODEL), jnp.float32),
        pltpu.SemaphoreType.DMA,
        pltpu.SemaphoreType.DMA,
    ],
)
def _sc_emit(x_hbm, proj_hbm, out_hbm, idx_v, proj_v, b0, b1, ws0, ws1):
    cid = lax.axis_index("c")
    sid = lax.axis_index("s")
    wid = sid * NC + cid
    base = wid * TW
    pltpu.sync_copy(x_hbm.at[pl.ds(base, TW)], idx_v)
    pltpu.sync_copy(proj_hbm, proj_v)

    bufs = (b0, b1)
    wsems = (ws0, ws1)
    lanes = lax.iota(jnp.int32, 16)

    def build(ci, p):
        xv = idx_v[pl.ds(ci * C, 16)]
        ws = []
        for i in range(C):
            si = jnp.sum(jnp.where(lanes == i, xv, 0))
            ws.append(jnp.full((16,), si, jnp.int32).astype(jnp.float32))

        def col(j, carry):
            o = j * 16
            p0 = proj_v[pl.ds(o, 16)]
            p1 = proj_v[pl.ds(D_MODEL + o, 16)]
            d = p1 - p0
            for i in range(C):
                bufs[p][i, pl.ds(o, 16)] = p0 + ws[i] * d
            return carry

        lax.fori_loop(0, D_MODEL // 16, col, 0, unroll=4)

    def start_write(ci, p):
        pltpu.async_copy(bufs[p], out_hbm.at[pl.ds(base + ci * C, C)], wsems[p])

    def wait_write(ci, p):
        pltpu.make_async_copy(
            bufs[p], out_hbm.at[pl.ds(base + ci * C, C)], wsems[p]
        ).wait()

    build(0, 0)

    def body(g, carry):
        ci0 = 2 * g
        start_write(ci0, 0)

        @pl.when(g > 0)
        def _():
            wait_write(ci0 - 1, 1)

        build(ci0 + 1, 1)
        start_write(ci0 + 1, 1)
        wait_write(ci0, 0)

        @pl.when(g < NG - 1)
        def _():
            build(ci0 + 2, 0)

        return carry

    lax.fori_loop(0, NG, body, 0)
    wait_write(NCHUNK - 1, 1)


def kernel(x, emb_table, W, b):
    proj = _compute_proj(emb_table, W, b)
    xf = x.reshape(-1).astype(jnp.int32)
    out = _sc_emit(xf, proj.reshape(-1))
    return out.reshape(B_DIM, L_DIM, D_MODEL)
